# Initial kernel scaffold; baseline (speedup 1.0000x reference)
import jax
import jax.numpy as jnp
from jax.experimental import pallas as pl


def kernel(Lnc_f_features, Gene_f_features, Lnc_f_edge_index, Gene_f_edge_index, W1l, b1l, W2l, b2l, W3l, b3l, W1g, b1g, W2g, b2g, W3g, b3g):
    raise NotImplementedError("write your pallas kernel here")


# speedup vs baseline: 9.0126x; 9.0126x over previous
import functools

import jax
import jax.numpy as jnp
from jax import lax
from jax.experimental import pallas as pl
from jax.experimental.pallas import tpu as pltpu
from jax.experimental.pallas import tpu_sc as plsc

N = 10000
E = 320000
D = 128
H = 128
C = 16

NTILES = 16
NP = 10240
ROWS_PER_TILE = NP // NTILES
EDGES_PER_TILE = E // NTILES
CHUNK = 80
NCHUNK = EDGES_PER_TILE // CHUNK

_sc_mesh = plsc.VectorSubcoreMesh(core_axis_name="c", subcore_axis_name="s")


@functools.partial(
    pl.kernel,
    out_type=jax.ShapeDtypeStruct((2 * NP,), jnp.float32),
    mesh=_sc_mesh,
    scratch_types=[
        pltpu.VMEM((CHUNK,), jnp.float32),
        pltpu.VMEM((CHUNK,), jnp.int32),
        pltpu.VMEM((ROWS_PER_TILE,), jnp.float32),
        pltpu.VMEM_SHARED((NP,), jnp.float32),
    ],
)
def _deg_kernel(dst_hbm, deg_hbm, ones_v, idx_v, stage_v, acc_sh):
    c = lax.axis_index("c")
    s = lax.axis_index("s")
    for j in range(CHUNK // 16):
        ones_v[pl.ds(j * 16, 16)] = jnp.ones((16,), jnp.float32)
    for j in range(ROWS_PER_TILE // 16):
        stage_v[pl.ds(j * 16, 16)] = jnp.zeros((16,), jnp.float32)
    pltpu.sync_copy(stage_v, acc_sh.at[pl.ds(s * ROWS_PER_TILE, ROWS_PER_TILE)])
    plsc.subcore_barrier()

    base = c * E + s * EDGES_PER_TILE

    def body(i, carry):
        pltpu.sync_copy(dst_hbm.at[pl.ds(base + i * CHUNK, CHUNK)], idx_v)
        pltpu.sync_copy(ones_v, acc_sh.at[idx_v], add=True)
        return carry

    lax.fori_loop(0, NCHUNK, body, 0)
    plsc.subcore_barrier()
    pltpu.sync_copy(acc_sh.at[pl.ds(s * ROWS_PER_TILE, ROWS_PER_TILE)], stage_v)
    pltpu.sync_copy(stage_v,
                    deg_hbm.at[pl.ds(c * NP + s * ROWS_PER_TILE, ROWS_PER_TILE)])


def _make_scatter(W):
    nstage = ROWS_PER_TILE // CHUNK

    @functools.partial(
        pl.kernel,
        out_type=jax.ShapeDtypeStruct((2, NP, W), jnp.float32),
        mesh=_sc_mesh,
        scratch_types=[
            pltpu.VMEM((CHUNK,), jnp.int32),
            pltpu.VMEM((CHUNK,), jnp.int32),
            pltpu.VMEM((CHUNK, W), jnp.float32),
            pltpu.VMEM_SHARED((NP, W), jnp.float32),
            pltpu.SemaphoreType.DMA,
        ],
    )
    def scat(hp_hbm, src_hbm, dst_hbm, zeros_hbm, agg_hbm,
             sidx, didx, rows, acc, sem):
        c = lax.axis_index("c")
        s = lax.axis_index("s")
        row0 = s * ROWS_PER_TILE
        pltpu.sync_copy(zeros_hbm, rows)
        for j in range(nstage):
            pltpu.sync_copy(rows, acc.at[pl.ds(row0 + j * CHUNK, CHUNK)])
        plsc.subcore_barrier()

        base = c * E + s * EDGES_PER_TILE

        def body(i, carry):
            e = base + i * CHUNK
            pltpu.sync_copy(src_hbm.at[pl.ds(e, CHUNK)], sidx)
            pltpu.sync_copy(dst_hbm.at[pl.ds(e, CHUNK)], didx)
            pltpu.async_copy(hp_hbm.at[sidx], rows, sem).wait()
            pltpu.sync_copy(rows, acc.at[didx], add=True)
            return carry

        lax.fori_loop(0, NCHUNK, body, 0)
        plsc.subcore_barrier()
        for j in range(nstage):
            r = row0 + j * CHUNK
            pltpu.sync_copy(acc.at[pl.ds(r, CHUNK)], rows)
            pltpu.sync_copy(rows, agg_hbm.at[c, pl.ds(r, CHUNK)])

    return scat


_scatter128 = _make_scatter(H)


def _mm_body(x_ref, w_ref, h_ref):
    for g in range(2):
        h_ref[g] = jnp.dot(x_ref[g], w_ref[g],
                           preferred_element_type=jnp.float32)


def _dinv_body(deg_ref, dinv_ref):
    dinv_ref[...] = lax.rsqrt(deg_ref[...] + 1.0)


def _scale_body(h_ref, d_ref, hp_ref):
    for g in range(2):
        hp_ref[g] = h_ref[g] * d_ref[g]


def _stage_body(agg_ref, hp_ref, d_ref, b_ref, w_ref, out_ref):
    for g in range(2):
        d = d_ref[g]
        a = d * (agg_ref[g, :N] + hp_ref[g]) + b_ref[g]
        a = jnp.maximum(a, 0.0)
        out_ref[g] = d * jnp.dot(a, w_ref[g],
                                 preferred_element_type=jnp.float32)


def _stage3_body(agg_ref, hp_ref, d_ref, b_ref, out_ref):
    for g in range(2):
        d = d_ref[g]
        a = d * (agg_ref[g, :N] + hp_ref[g]) + b_ref[g]
        out_ref[g] = d * jnp.maximum(a, 0.0)


def _final_body(agg_ref, hq_ref, d_ref, b_ref, w_ref, out_ref):
    for g in range(2):
        a = agg_ref[g, :N] + hq_ref[g]
        out_ref[g] = d_ref[g] * jnp.dot(
            a, w_ref[g], preferred_element_type=jnp.float32) + b_ref[g]


def _tc(body, out_shape, *args):
    return pl.pallas_call(body, out_shape=out_shape)(*args)


def kernel(Lnc_f_features, Gene_f_features, Lnc_f_edge_index,
           Gene_f_edge_index, W1l, b1l, W2l, b2l, W3l, b3l,
           W1g, b1g, W2g, b2g, W3g, b3g):
    f32 = jnp.float32
    x = jnp.stack([Lnc_f_features, Gene_f_features])
    src = jnp.concatenate([Lnc_f_edge_index[0],
                           Gene_f_edge_index[0] + N])
    dst = jnp.concatenate([Lnc_f_edge_index[1], Gene_f_edge_index[1]])
    W1 = jnp.stack([W1l, W1g])
    W2 = jnp.stack([W2l, W2g])
    W3 = jnp.stack([W3l, W3g])
    b1 = jnp.stack([b1l, b1g])[:, None, :]
    b2 = jnp.stack([b2l, b2g])[:, None, :]
    b3 = jnp.stack([b3l, b3g])[:, None, :]
    zeros_h = jnp.zeros((CHUNK, H), f32)

    deg = _deg_kernel(dst).reshape(2, NP)
    h1 = _tc(_mm_body, jax.ShapeDtypeStruct((2, N, H), f32), x, W1)
    dinv = _tc(_dinv_body, jax.ShapeDtypeStruct((2, NP), f32), deg)
    dinvc = dinv[:, :N, None]

    hp1 = _tc(_scale_body, jax.ShapeDtypeStruct((2, N, H), f32), h1, dinvc)
    agg1 = _scatter128(hp1.reshape(2 * N, H), src, dst, zeros_h)
    hp2 = _tc(_stage_body, jax.ShapeDtypeStruct((2, N, H), f32),
              agg1, hp1, dinvc, b1, W2)
    agg2 = _scatter128(hp2.reshape(2 * N, H), src, dst, zeros_h)
    hq = _tc(_stage3_body, jax.ShapeDtypeStruct((2, N, H), f32),
             agg2, hp2, dinvc, b2)
    agg3 = _scatter128(hq.reshape(2 * N, H), src, dst, zeros_h)
    out = _tc(_final_body, jax.ShapeDtypeStruct((2, N, C), f32),
              agg3, hq, dinvc, b3, W3)
    return out[0], out[1]


# speedup vs baseline: 23.1507x; 2.5687x over previous
import functools

import jax
import jax.numpy as jnp
from jax import lax
from jax.experimental import pallas as pl
from jax.experimental.pallas import tpu as pltpu
from jax.experimental.pallas import tpu_sc as plsc

N = 10000
E = 320000
D = 128
H = 128
C = 16

NTILES = 16
NP = 10240
ROWS_PER_TILE = NP // NTILES
CHUNK = 128
EP = 327680
ECHUNKS = 2 * EP // CHUNK
TILE_ECHUNKS = EP // (NTILES * CHUNK)
BLK = 16
NBLK = TILE_ECHUNKS // BLK

_sc_mesh = plsc.VectorSubcoreMesh(core_axis_name="c", subcore_axis_name="s")


@functools.partial(
    pl.kernel,
    out_type=jax.ShapeDtypeStruct((2 * NP,), jnp.float32),
    mesh=_sc_mesh,
    scratch_types=[
        pltpu.VMEM((CHUNK,), jnp.float32),
        pltpu.VMEM((BLK, CHUNK), jnp.int32),
        pltpu.VMEM((ROWS_PER_TILE,), jnp.float32),
        pltpu.VMEM_SHARED((NP,), jnp.float32),
    ],
)
def _deg_kernel(dst_hbm, deg_hbm, ones_v, dblk, stage_v, acc_sh):
    c = lax.axis_index("c")
    s = lax.axis_index("s")
    for j in range(CHUNK // 16):
        ones_v[pl.ds(j * 16, 16)] = jnp.ones((16,), jnp.float32)
    for j in range(ROWS_PER_TILE // 16):
        stage_v[pl.ds(j * 16, 16)] = jnp.zeros((16,), jnp.float32)
    pltpu.sync_copy(stage_v, acc_sh.at[pl.ds(s * ROWS_PER_TILE, ROWS_PER_TILE)])
    plsc.subcore_barrier()

    r0 = c * (ECHUNKS // 2) + s * TILE_ECHUNKS

    def body(i, carry):
        pltpu.sync_copy(dst_hbm.at[pl.ds(r0 + i * BLK, BLK)], dblk)
        for j in range(BLK):
            pltpu.sync_copy(ones_v, acc_sh.at[dblk.at[j]], add=True)
        return carry

    lax.fori_loop(0, NBLK, body, 0)
    plsc.subcore_barrier()
    pltpu.sync_copy(acc_sh.at[pl.ds(s * ROWS_PER_TILE, ROWS_PER_TILE)], stage_v)
    pltpu.sync_copy(stage_v,
                    deg_hbm.at[pl.ds(c * NP + s * ROWS_PER_TILE, ROWS_PER_TILE)])


def _make_scatter(W):
    nstage = ROWS_PER_TILE // CHUNK

    @functools.partial(
        pl.kernel,
        out_type=jax.ShapeDtypeStruct((2, NP, W), jnp.float32),
        mesh=_sc_mesh,
        scratch_types=[
            pltpu.VMEM((BLK, CHUNK), jnp.int32),
            pltpu.VMEM((BLK, CHUNK), jnp.int32),
            pltpu.VMEM((2, CHUNK, W), jnp.float32),
            pltpu.VMEM_SHARED((NP, W), jnp.float32),
            pltpu.SemaphoreType.DMA,
        ],
    )
    def scat(hp_hbm, src_hbm, dst_hbm, zeros_hbm, agg_hbm,
             sblk, dblk, rows, acc, gsem):
        c = lax.axis_index("c")
        s = lax.axis_index("s")
        row0 = s * ROWS_PER_TILE
        pltpu.sync_copy(zeros_hbm, rows.at[0])
        for j in range(nstage):
            pltpu.sync_copy(rows.at[0], acc.at[pl.ds(row0 + j * CHUNK, CHUNK)])
        plsc.subcore_barrier()

        r0 = c * (ECHUNKS // 2) + s * TILE_ECHUNKS

        def body(i, carry):
            pltpu.sync_copy(src_hbm.at[pl.ds(r0 + i * BLK, BLK)], sblk)
            pltpu.sync_copy(dst_hbm.at[pl.ds(r0 + i * BLK, BLK)], dblk)
            desc = pltpu.async_copy(hp_hbm.at[sblk.at[0]], rows.at[0], gsem)
            for j in range(BLK):
                b = j & 1
                if j + 1 < BLK:
                    nxt = pltpu.async_copy(hp_hbm.at[sblk.at[j + 1]],
                                           rows.at[b ^ 1], gsem)
                desc.wait()
                pltpu.sync_copy(rows.at[b], acc.at[dblk.at[j]], add=True)
                if j + 1 < BLK:
                    desc = nxt
            return carry

        lax.fori_loop(0, NBLK, body, 0)
        plsc.subcore_barrier()
        for j in range(nstage):
            r = row0 + j * CHUNK
            pltpu.sync_copy(acc.at[pl.ds(r, CHUNK)], rows.at[0])
            pltpu.sync_copy(rows.at[0], agg_hbm.at[c, pl.ds(r, CHUNK)])

    return scat


_scatter128 = _make_scatter(H)


def _mm_body(x_ref, w_ref, h_ref):
    for g in range(2):
        h_ref[g] = jnp.dot(x_ref[g], w_ref[g],
                           preferred_element_type=jnp.float32)


def _dinv_body(deg_ref, dinv_ref):
    dinv_ref[...] = lax.rsqrt(deg_ref[...] + 1.0)


def _scale_body(h_ref, d_ref, hp_ref):
    for g in range(2):
        hp_ref[g] = h_ref[g] * d_ref[g]


def _stage_body(agg_ref, hp_ref, d_ref, b_ref, w_ref, out_ref):
    for g in range(2):
        d = d_ref[g]
        a = d * (agg_ref[g, :N] + hp_ref[g]) + b_ref[g]
        a = jnp.maximum(a, 0.0)
        out_ref[g] = d * jnp.dot(a, w_ref[g],
                                 preferred_element_type=jnp.float32)


def _stage3_body(agg_ref, hp_ref, d_ref, b_ref, out_ref):
    for g in range(2):
        d = d_ref[g]
        a = d * (agg_ref[g, :N] + hp_ref[g]) + b_ref[g]
        out_ref[g] = d * jnp.maximum(a, 0.0)


def _final_body(agg_ref, hq_ref, d_ref, b_ref, w_ref, out_ref):
    for g in range(2):
        a = agg_ref[g, :N] + hq_ref[g]
        out_ref[g] = d_ref[g] * jnp.dot(
            a, w_ref[g], preferred_element_type=jnp.float32) + b_ref[g]


def _tc(body, out_shape, *args):
    return pl.pallas_call(body, out_shape=out_shape)(*args)


def kernel(Lnc_f_features, Gene_f_features, Lnc_f_edge_index,
           Gene_f_edge_index, W1l, b1l, W2l, b2l, W3l, b3l,
           W1g, b1g, W2g, b2g, W3g, b3g):
    f32 = jnp.float32
    x = jnp.stack([Lnc_f_features, Gene_f_features])
    npad = EP - E
    pad_src = (jnp.arange(npad, dtype=jnp.int32) * 131) % N
    pad_dst = N + (jnp.arange(npad, dtype=jnp.int32) % (NP - N))
    src = jnp.concatenate([Lnc_f_edge_index[0], pad_src,
                           Gene_f_edge_index[0] + N, pad_src + N])
    src = src.reshape(ECHUNKS, CHUNK)
    dst = jnp.concatenate([Lnc_f_edge_index[1], pad_dst,
                           Gene_f_edge_index[1], pad_dst])
    dst = dst.reshape(ECHUNKS, CHUNK)
    W1 = jnp.stack([W1l, W1g])
    W2 = jnp.stack([W2l, W2g])
    W3 = jnp.stack([W3l, W3g])
    b1 = jnp.stack([b1l, b1g])[:, None, :]
    b2 = jnp.stack([b2l, b2g])[:, None, :]
    b3 = jnp.stack([b3l, b3g])[:, None, :]
    zeros_h = jnp.zeros((CHUNK, H), f32)

    deg = _deg_kernel(dst).reshape(2, NP)
    h1 = _tc(_mm_body, jax.ShapeDtypeStruct((2, N, H), f32), x, W1)
    dinv = _tc(_dinv_body, jax.ShapeDtypeStruct((2, NP), f32), deg)
    dinvc = dinv[:, :N, None]

    hp1 = _tc(_scale_body, jax.ShapeDtypeStruct((2, N, H), f32), h1, dinvc)
    agg1 = _scatter128(hp1.reshape(2 * N, H), src, dst, zeros_h)
    hp2 = _tc(_stage_body, jax.ShapeDtypeStruct((2, N, H), f32),
              agg1, hp1, dinvc, b1, W2)
    agg2 = _scatter128(hp2.reshape(2 * N, H), src, dst, zeros_h)
    hq = _tc(_stage3_body, jax.ShapeDtypeStruct((2, N, H), f32),
             agg2, hp2, dinvc, b2)
    agg3 = _scatter128(hq.reshape(2 * N, H), src, dst, zeros_h)
    out = _tc(_final_body, jax.ShapeDtypeStruct((2, N, C), f32),
              agg3, hq, dinvc, b3, W3)
    return out[0], out[1]


# speedup vs baseline: 23.4550x; 1.0131x over previous
import functools

import jax
import jax.numpy as jnp
from jax import lax
from jax.experimental import pallas as pl
from jax.experimental.pallas import tpu as pltpu
from jax.experimental.pallas import tpu_sc as plsc

N = 10000
E = 320000
D = 128
H = 128
C = 16

NTILES = 16
NP = 10240
ROWS_PER_TILE = NP // NTILES
CHUNK = 128
EP = 327680
ECHUNKS = 2 * EP // CHUNK
TILE_ECHUNKS = EP // (NTILES * CHUNK)
BLK = 16
NBLK = TILE_ECHUNKS // BLK

_sc_mesh = plsc.VectorSubcoreMesh(core_axis_name="c", subcore_axis_name="s")


@functools.partial(
    pl.kernel,
    out_type=jax.ShapeDtypeStruct((2 * NP,), jnp.float32),
    mesh=_sc_mesh,
    scratch_types=[
        pltpu.VMEM((CHUNK,), jnp.float32),
        pltpu.VMEM((BLK, CHUNK), jnp.int32),
        pltpu.VMEM((ROWS_PER_TILE,), jnp.float32),
        pltpu.VMEM_SHARED((NP,), jnp.float32),
        pltpu.SemaphoreType.DMA,
    ],
)
def _deg_kernel(dst_hbm, deg_hbm, ones_v, dblk, stage_v, acc_sh, dsem):
    c = lax.axis_index("c")
    s = lax.axis_index("s")
    for j in range(CHUNK // 16):
        ones_v[pl.ds(j * 16, 16)] = jnp.ones((16,), jnp.float32)
    for j in range(ROWS_PER_TILE // 16):
        stage_v[pl.ds(j * 16, 16)] = jnp.zeros((16,), jnp.float32)
    pltpu.sync_copy(stage_v, acc_sh.at[pl.ds(s * ROWS_PER_TILE, ROWS_PER_TILE)])
    plsc.subcore_barrier()

    r0 = c * (ECHUNKS // 2) + s * TILE_ECHUNKS

    def body(i, carry):
        pltpu.sync_copy(dst_hbm.at[pl.ds(r0 + i * BLK, BLK)], dblk)
        descs = [pltpu.async_copy(ones_v, acc_sh.at[dblk.at[j]], dsem,
                                  add=True) for j in range(BLK)]
        for d in descs:
            d.wait()
        return carry

    lax.fori_loop(0, NBLK, body, 0)
    plsc.subcore_barrier()
    pltpu.sync_copy(acc_sh.at[pl.ds(s * ROWS_PER_TILE, ROWS_PER_TILE)], stage_v)
    pltpu.sync_copy(stage_v,
                    deg_hbm.at[pl.ds(c * NP + s * ROWS_PER_TILE, ROWS_PER_TILE)])


def _make_scatter(W):
    nstage = ROWS_PER_TILE // CHUNK

    @functools.partial(
        pl.kernel,
        out_type=jax.ShapeDtypeStruct((2, NP, W), jnp.float32),
        mesh=_sc_mesh,
        scratch_types=[
            pltpu.VMEM((BLK, CHUNK), jnp.int32),
            pltpu.VMEM((BLK, CHUNK), jnp.int32),
            pltpu.VMEM((2, CHUNK, W), jnp.float32),
            pltpu.VMEM_SHARED((NP, W), jnp.float32),
            pltpu.SemaphoreType.DMA,
            pltpu.SemaphoreType.DMA,
        ],
    )
    def scat(hp_hbm, src_hbm, dst_hbm, zeros_hbm, agg_hbm,
             sblk, dblk, rows, acc, gsem, ssem):
        c = lax.axis_index("c")
        s = lax.axis_index("s")
        row0 = s * ROWS_PER_TILE
        pltpu.sync_copy(zeros_hbm, rows.at[0])
        for j in range(nstage):
            pltpu.sync_copy(rows.at[0], acc.at[pl.ds(row0 + j * CHUNK, CHUNK)])
        plsc.subcore_barrier()

        r0 = c * (ECHUNKS // 2) + s * TILE_ECHUNKS

        def body(i, carry):
            pltpu.sync_copy(src_hbm.at[pl.ds(r0 + i * BLK, BLK)], sblk)
            pltpu.sync_copy(dst_hbm.at[pl.ds(r0 + i * BLK, BLK)], dblk)
            desc = pltpu.async_copy(hp_hbm.at[sblk.at[0]], rows.at[0], gsem)
            for j in range(BLK):
                b = j & 1
                if j + 1 < BLK:
                    nxt = pltpu.async_copy(hp_hbm.at[sblk.at[j + 1]],
                                           rows.at[b ^ 1], gsem)
                desc.wait()
                pltpu.sync_copy(rows.at[b], acc.at[dblk.at[j]], add=True)
                if j + 1 < BLK:
                    desc = nxt
            return carry

        lax.fori_loop(0, NBLK, body, 0)
        plsc.subcore_barrier()
        for j in range(nstage):
            r = row0 + j * CHUNK
            pltpu.sync_copy(acc.at[pl.ds(r, CHUNK)], rows.at[0])
            pltpu.sync_copy(rows.at[0], agg_hbm.at[c, pl.ds(r, CHUNK)])

    return scat


_scatter128 = _make_scatter(H)


def _prep_body(x_ref, w_ref, degc_ref, hp_ref, dinv_ref):
    for g in range(2):
        d = lax.rsqrt(degc_ref[g] + 1.0)
        dinv_ref[g] = d
        hp_ref[g] = d * jnp.dot(x_ref[g], w_ref[g],
                                preferred_element_type=jnp.float32)


def _stage_body(agg_ref, hp_ref, d_ref, b_ref, w_ref, out_ref):
    for g in range(2):
        d = d_ref[g]
        a = d * (agg_ref[g, :N] + hp_ref[g]) + b_ref[g]
        a = jnp.maximum(a, 0.0)
        out_ref[g] = d * jnp.dot(a, w_ref[g],
                                 preferred_element_type=jnp.float32)


def _stage3_body(agg_ref, hp_ref, d_ref, b_ref, out_ref):
    for g in range(2):
        d = d_ref[g]
        a = d * (agg_ref[g, :N] + hp_ref[g]) + b_ref[g]
        out_ref[g] = d * jnp.maximum(a, 0.0)


def _final_body(agg_ref, hq_ref, d_ref, b_ref, w_ref, out_ref):
    for g in range(2):
        a = agg_ref[g, :N] + hq_ref[g]
        out_ref[g] = d_ref[g] * jnp.dot(
            a, w_ref[g], preferred_element_type=jnp.float32) + b_ref[g]


def _tc(body, out_shape, *args):
    return pl.pallas_call(body, out_shape=out_shape)(*args)


def kernel(Lnc_f_features, Gene_f_features, Lnc_f_edge_index,
           Gene_f_edge_index, W1l, b1l, W2l, b2l, W3l, b3l,
           W1g, b1g, W2g, b2g, W3g, b3g):
    f32 = jnp.float32
    x = jnp.stack([Lnc_f_features, Gene_f_features])
    npad = EP - E
    pad_src = (jnp.arange(npad, dtype=jnp.int32) * 131) % N
    pad_dst = N + (jnp.arange(npad, dtype=jnp.int32) % (NP - N))
    src = jnp.concatenate([Lnc_f_edge_index[0], pad_src,
                           Gene_f_edge_index[0] + N, pad_src + N])
    src = src.reshape(ECHUNKS, CHUNK)
    dst = jnp.concatenate([Lnc_f_edge_index[1], pad_dst,
                           Gene_f_edge_index[1], pad_dst])
    dst = dst.reshape(ECHUNKS, CHUNK)
    W1 = jnp.stack([W1l, W1g])
    W2 = jnp.stack([W2l, W2g])
    W3 = jnp.stack([W3l, W3g])
    b1 = jnp.stack([b1l, b1g])[:, None, :]
    b2 = jnp.stack([b2l, b2g])[:, None, :]
    b3 = jnp.stack([b3l, b3g])[:, None, :]
    zeros_h = jnp.zeros((CHUNK, H), f32)

    deg = _deg_kernel(dst).reshape(2, NP)
    degc = deg[:, :N, None]
    hp1, dinvc = _tc(_prep_body,
                     (jax.ShapeDtypeStruct((2, N, H), f32),
                      jax.ShapeDtypeStruct((2, N, 1), f32)),
                     x, W1, degc)
    agg1 = _scatter128(hp1.reshape(2 * N, H), src, dst, zeros_h)
    hp2 = _tc(_stage_body, jax.ShapeDtypeStruct((2, N, H), f32),
              agg1, hp1, dinvc, b1, W2)
    agg2 = _scatter128(hp2.reshape(2 * N, H), src, dst, zeros_h)
    hq = _tc(_stage3_body, jax.ShapeDtypeStruct((2, N, H), f32),
             agg2, hp2, dinvc, b2)
    agg3 = _scatter128(hq.reshape(2 * N, H), src, dst, zeros_h)
    out = _tc(_final_body, jax.ShapeDtypeStruct((2, N, C), f32),
              agg3, hq, dinvc, b3, W3)
    return out[0], out[1]
